# baseline (speedup 1.0000x reference)
import functools

import jax
import jax.numpy as jnp
from jax import lax
from jax.experimental import pallas as pl
from jax.experimental.pallas import tpu as pltpu
from jax.experimental.pallas import tpu_sc as plsc

E = 8
K = 2
D = 2048
H = 4096
T = 2 * 2048
A = T * K
BM = 512
G = A // BM + E
P = G * BM
HC = 8
Hc = H // HC

NC, NS = 2, 16
NW = NC * NS
CH = 32


def _router_body(x_ref, gw_ref, ids_ref, wts_ref):
    xb = x_ref[...]
    logits = lax.dot_general(xb, gw_ref[...], (((1,), (1,)), ((), ())),
                             preferred_element_type=jnp.float32)
    m = jnp.max(logits, axis=1, keepdims=True)
    ex = jnp.exp(logits - m)
    p = ex / jnp.sum(ex, axis=1, keepdims=True)
    cols = lax.broadcasted_iota(jnp.int32, p.shape, 1)
    m1 = jnp.max(p, axis=1)
    a1 = jnp.min(jnp.where(p >= m1[:, None], cols, E), axis=1)
    pm = jnp.where(cols == a1[:, None], jnp.float32(-1), p)
    m2 = jnp.max(pm, axis=1)
    a2 = jnp.min(jnp.where(pm >= m2[:, None], cols, E), axis=1)
    s = m1 + m2
    ids_ref[...] = jnp.concatenate([a1[:, None], a2[:, None]], axis=1)
    wts_ref[...] = jnp.concatenate([(m1 / s)[:, None], (m2 / s)[:, None]],
                                   axis=1)


def _router(x, gate_w):
    bt = 512
    return pl.pallas_call(
        _router_body,
        grid=(T // bt,),
        in_specs=[
            pl.BlockSpec((bt, D), lambda i: (i, 0)),
            pl.BlockSpec((E, D), lambda i: (0, 0)),
        ],
        out_specs=[
            pl.BlockSpec((bt, K), lambda i: (i, 0)),
            pl.BlockSpec((bt, K), lambda i: (i, 0)),
        ],
        out_shape=[
            jax.ShapeDtypeStruct((T, K), jnp.int32),
            jax.ShapeDtypeStruct((T, K), jnp.float32),
        ],
    )(x, gate_w)


def _make_row_gather(n_src, n_out):
    rpw = n_out // NW
    iters = rpw // CH
    mesh = plsc.VectorSubcoreMesh(core_axis_name="c", subcore_axis_name="s",
                                  num_cores=NC, num_subcores=NS)

    @functools.partial(
        pl.kernel,
        mesh=mesh,
        out_type=jax.ShapeDtypeStruct((n_out, D), jnp.float32),
        scratch_types=[
            pltpu.VMEM((CH,), jnp.int32),
            pltpu.VMEM((CH, D), jnp.float32),
            pltpu.SemaphoreType.DMA,
        ],
    )
    def gather_k(src_hbm, idx_hbm, out_hbm, idx_v, rows_v, sem):
        wid = lax.axis_index("s") * NC + lax.axis_index("c")
        base0 = wid * rpw
        for i in range(iters):
            base = base0 + i * CH
            pltpu.sync_copy(idx_hbm.at[pl.ds(base, CH)], idx_v)
            pltpu.async_copy(src_hbm.at[idx_v], rows_v, sem).wait()
            pltpu.sync_copy(rows_v, out_hbm.at[pl.ds(base, CH)])

    return gather_k


_gather_x = None
_gather_y = None


def _get_gathers():
    global _gather_x, _gather_y
    if _gather_x is None:
        _gather_x = _make_row_gather(T, P)
        _gather_y = _make_row_gather(P, A)
    return _gather_x, _gather_y


def _ffn_body(be_ref, x_ref, w1_ref, w3_ref, w2_ref, out_ref, acc_ref):
    del be_ref
    hc = pl.program_id(1)
    xb = x_ref[...].astype(jnp.bfloat16)
    a1 = lax.dot_general(xb, w1_ref[0], (((1,), (1,)), ((), ())),
                         preferred_element_type=jnp.float32)
    a3 = lax.dot_general(xb, w3_ref[0], (((1,), (1,)), ((), ())),
                         preferred_element_type=jnp.float32)
    h = (a1 * jax.nn.sigmoid(a1) * a3).astype(jnp.bfloat16)
    part = lax.dot_general(h, w2_ref[0], (((1,), (1,)), ((), ())),
                           preferred_element_type=jnp.float32)

    @pl.when(hc == 0)
    def _():
        acc_ref[...] = part

    @pl.when(hc > 0)
    def _():
        acc_ref[...] = acc_ref[...] + part

    @pl.when(hc == HC - 1)
    def _():
        out_ref[...] = acc_ref[...]


def _grouped_ffn(block_expert, x_sorted, w1, w3, w2):
    grid_spec = pltpu.PrefetchScalarGridSpec(
        num_scalar_prefetch=1,
        grid=(G, HC),
        in_specs=[
            pl.BlockSpec((BM, D), lambda g, hc, be: (g, 0)),
            pl.BlockSpec((1, Hc, D), lambda g, hc, be: (be[g], hc, 0)),
            pl.BlockSpec((1, Hc, D), lambda g, hc, be: (be[g], hc, 0)),
            pl.BlockSpec((1, D, Hc), lambda g, hc, be: (be[g], 0, hc)),
        ],
        out_specs=pl.BlockSpec((BM, D), lambda g, hc, be: (g, 0)),
        scratch_shapes=[pltpu.VMEM((BM, D), jnp.float32)],
    )
    return pl.pallas_call(
        _ffn_body,
        grid_spec=grid_spec,
        out_shape=jax.ShapeDtypeStruct((P, D), jnp.float32),
        compiler_params=pltpu.CompilerParams(
            dimension_semantics=("arbitrary", "arbitrary"),
        ),
    )(block_expert, x_sorted, w1, w3, w2)


def _combine_body(y_ref, w_ref, o_ref):
    w0 = w_ref[:, 0:1]
    w1c = w_ref[:, 1:2]
    o_ref[...] = y_ref[:, :D] * w0 + y_ref[:, D:] * w1c


def _combine(y, wts):
    bt = 512
    return pl.pallas_call(
        _combine_body,
        grid=(T // bt,),
        in_specs=[
            pl.BlockSpec((bt, K * D), lambda i: (i, 0)),
            pl.BlockSpec((bt, K), lambda i: (i, 0)),
        ],
        out_specs=pl.BlockSpec((bt, D), lambda i: (i, 0)),
        out_shape=jax.ShapeDtypeStruct((T, D), jnp.float32),
    )(y, wts)


def kernel(hidden_states, gate_w, w1, w2, w3):
    orig_shape = hidden_states.shape
    x = hidden_states.reshape(T, D)

    ids, wts = _router(x, gate_w)

    flat_e = ids.reshape(-1)
    oh = (flat_e[:, None] == jnp.arange(E, dtype=jnp.int32)[None, :]
          ).astype(jnp.int32)
    csum = jnp.cumsum(oh, axis=0)
    counts = csum[-1]
    local = jnp.take_along_axis(csum, flat_e[:, None], axis=1)[:, 0] - 1
    padded = ((counts + BM - 1) // BM) * BM
    pcs = jnp.cumsum(padded)
    poff = pcs - padded
    pos = poff[flat_e] + local
    tok_src = (jnp.arange(P, dtype=jnp.int32) % T).at[pos].set(
        jnp.arange(A, dtype=jnp.int32) // K)
    inv_pos = pos
    block_expert = jnp.clip(
        jnp.searchsorted(pcs, jnp.arange(G, dtype=jnp.int32) * BM,
                         side="right"),
        0, E - 1).astype(jnp.int32)

    gather_x, gather_y = _get_gathers()
    x_sorted = gather_x(x, tok_src)
    contrib = _grouped_ffn(block_expert, x_sorted,
                           w1.astype(jnp.bfloat16), w3.astype(jnp.bfloat16),
                           w2.astype(jnp.bfloat16))
    y = gather_y(contrib, inv_pos).reshape(T, K * D)
    out = _combine(y, wts)
    return out.reshape(orig_shape)


# speedup vs baseline: 1.4571x; 1.4571x over previous
import functools

import jax
import jax.numpy as jnp
from jax import lax
from jax.experimental import pallas as pl
from jax.experimental.pallas import tpu as pltpu
from jax.experimental.pallas import tpu_sc as plsc

E = 8
K = 2
D = 2048
H = 4096
T = 2 * 2048
A = T * K
BM = 512
G = A // BM + E
P = G * BM
HC = 8
Hc = H // HC

NC, NS = 2, 16
NW = NC * NS
CH = 32


def _router_body(x_ref, gw_ref, ids_ref, wts_ref):
    xb = x_ref[...]
    logits = lax.dot_general(xb, gw_ref[...], (((1,), (1,)), ((), ())),
                             preferred_element_type=jnp.float32)
    m = jnp.max(logits, axis=1, keepdims=True)
    ex = jnp.exp(logits - m)
    p = ex / jnp.sum(ex, axis=1, keepdims=True)
    cols = lax.broadcasted_iota(jnp.int32, p.shape, 1)
    m1 = jnp.max(p, axis=1)
    a1 = jnp.min(jnp.where(p >= m1[:, None], cols, E), axis=1)
    pm = jnp.where(cols == a1[:, None], jnp.float32(-1), p)
    m2 = jnp.max(pm, axis=1)
    a2 = jnp.min(jnp.where(pm >= m2[:, None], cols, E), axis=1)
    s = m1 + m2
    ids_ref[...] = jnp.concatenate([a1[:, None], a2[:, None]], axis=1)
    wts_ref[...] = jnp.concatenate([(m1 / s)[:, None], (m2 / s)[:, None]],
                                   axis=1)


def _router(x, gate_w):
    bt = 512
    return pl.pallas_call(
        _router_body,
        grid=(T // bt,),
        in_specs=[
            pl.BlockSpec((bt, D), lambda i: (i, 0)),
            pl.BlockSpec((E, D), lambda i: (0, 0)),
        ],
        out_specs=[
            pl.BlockSpec((bt, K), lambda i: (i, 0)),
            pl.BlockSpec((bt, K), lambda i: (i, 0)),
        ],
        out_shape=[
            jax.ShapeDtypeStruct((T, K), jnp.int32),
            jax.ShapeDtypeStruct((T, K), jnp.float32),
        ],
    )(x, gate_w)


def _make_row_gather(n_src, n_out, ch):
    rpw = n_out // NW
    iters = rpw // ch
    mesh = plsc.VectorSubcoreMesh(core_axis_name="c", subcore_axis_name="s",
                                  num_cores=NC, num_subcores=NS)

    @functools.partial(
        pl.kernel,
        mesh=mesh,
        out_type=jax.ShapeDtypeStruct((n_out, D), jnp.float32),
        scratch_types=[
            pltpu.VMEM((rpw,), jnp.int32),
            pltpu.VMEM((ch, D), jnp.float32),
            pltpu.VMEM((ch, D), jnp.float32),
            pltpu.SemaphoreType.DMA,
            pltpu.SemaphoreType.DMA,
            pltpu.SemaphoreType.DMA,
            pltpu.SemaphoreType.DMA,
        ],
    )
    def gather_k(src_hbm, idx_hbm, out_hbm, idx_all, buf0, buf1,
                 gs0, gs1, ws0, ws1):
        wid = lax.axis_index("s") * NC + lax.axis_index("c")
        base0 = wid * rpw
        pltpu.sync_copy(idx_hbm.at[pl.ds(base0, rpw)], idx_all)
        bufs = (buf0, buf1)
        gsems = (gs0, gs1)
        wsems = (ws0, ws1)

        def start_gather(i):
            return pltpu.async_copy(
                src_hbm.at[idx_all.at[pl.ds(i * ch, ch)]],
                bufs[i % 2], gsems[i % 2])

        def start_wb(i):
            return pltpu.async_copy(
                bufs[i % 2], out_hbm.at[pl.ds(base0 + i * ch, ch)],
                wsems[i % 2])

        g = [None] * iters
        w = [None] * iters
        g[0] = start_gather(0)
        for i in range(iters):
            if i + 1 < iters:
                if i >= 1:
                    w[i - 1].wait()
                g[i + 1] = start_gather(i + 1)
            g[i].wait()
            w[i] = start_wb(i)
        if iters >= 2:
            w[iters - 2].wait()
        w[iters - 1].wait()

    return gather_k


_gather_x = None
_gather_y = None


def _get_gathers():
    global _gather_x, _gather_y
    if _gather_x is None:
        _gather_x = _make_row_gather(T, P, 24)
        _gather_y = _make_row_gather(P, A, 16)
    return _gather_x, _gather_y


def _ffn_body(be_ref, nu_ref, x_ref, w1_ref, w3_ref, w2_ref, out_ref, acc_ref):
    del be_ref
    g = pl.program_id(0)
    hc = pl.program_id(1)

    @pl.when(g < nu_ref[0])
    def _():
        xb = x_ref[...]
        a1 = lax.dot_general(xb, w1_ref[0], (((1,), (1,)), ((), ())),
                             preferred_element_type=jnp.float32)
        a3 = lax.dot_general(xb, w3_ref[0], (((1,), (1,)), ((), ())),
                             preferred_element_type=jnp.float32)
        h = a1 * jax.nn.sigmoid(a1) * a3
        part = lax.dot_general(h, w2_ref[0], (((1,), (1,)), ((), ())),
                               preferred_element_type=jnp.float32)

        @pl.when(hc == 0)
        def _():
            acc_ref[...] = part

        @pl.when(hc > 0)
        def _():
            acc_ref[...] = acc_ref[...] + part

        @pl.when(hc == HC - 1)
        def _():
            out_ref[...] = acc_ref[...]


def _grouped_ffn(block_expert, nused, x_sorted, w1, w3, w2):
    def wmap(g, hc, be, nu):
        return (be[g], jnp.where(g < nu[0], hc, HC - 1), 0)

    def w2map(g, hc, be, nu):
        return (be[g], 0, jnp.where(g < nu[0], hc, HC - 1))

    grid_spec = pltpu.PrefetchScalarGridSpec(
        num_scalar_prefetch=2,
        grid=(G, HC),
        in_specs=[
            pl.BlockSpec((BM, D),
                         lambda g, hc, be, nu: (jnp.minimum(g, nu[0] - 1), 0)),
            pl.BlockSpec((1, Hc, D), wmap),
            pl.BlockSpec((1, Hc, D), wmap),
            pl.BlockSpec((1, D, Hc), w2map),
        ],
        out_specs=pl.BlockSpec((BM, D), lambda g, hc, be, nu: (g, 0)),
        scratch_shapes=[pltpu.VMEM((BM, D), jnp.float32)],
    )
    return pl.pallas_call(
        _ffn_body,
        grid_spec=grid_spec,
        out_shape=jax.ShapeDtypeStruct((P, D), jnp.float32),
        compiler_params=pltpu.CompilerParams(
            dimension_semantics=("arbitrary", "arbitrary"),
            vmem_limit_bytes=100 * 1024 * 1024,
        ),
    )(block_expert, nused, x_sorted, w1, w3, w2)


def _combine_body(y_ref, w_ref, o_ref):
    w0 = w_ref[:, 0:1]
    w1c = w_ref[:, 1:2]
    o_ref[...] = y_ref[:, :D] * w0 + y_ref[:, D:] * w1c


def _combine(y, wts):
    bt = 512
    return pl.pallas_call(
        _combine_body,
        grid=(T // bt,),
        in_specs=[
            pl.BlockSpec((bt, K * D), lambda i: (i, 0)),
            pl.BlockSpec((bt, K), lambda i: (i, 0)),
        ],
        out_specs=pl.BlockSpec((bt, D), lambda i: (i, 0)),
        out_shape=jax.ShapeDtypeStruct((T, D), jnp.float32),
    )(y, wts)


def kernel(hidden_states, gate_w, w1, w2, w3):
    orig_shape = hidden_states.shape
    x = hidden_states.reshape(T, D)

    ids, wts = _router(x, gate_w)

    flat_e = ids.reshape(-1)
    oh = (flat_e[:, None] == jnp.arange(E, dtype=jnp.int32)[None, :]
          ).astype(jnp.int32)
    csum = jnp.cumsum(oh, axis=0)
    counts = csum[-1]
    local = jnp.take_along_axis(csum, flat_e[:, None], axis=1)[:, 0] - 1
    padded = ((counts + BM - 1) // BM) * BM
    pcs = jnp.cumsum(padded)
    poff = pcs - padded
    pos = poff[flat_e] + local
    tok_src = (jnp.arange(P, dtype=jnp.int32) % T).at[pos].set(
        jnp.arange(A, dtype=jnp.int32) // K)
    inv_pos = pos
    nused = (pcs[-1] // BM).astype(jnp.int32)
    block_expert = jnp.clip(
        jnp.searchsorted(pcs, jnp.minimum(jnp.arange(G, dtype=jnp.int32),
                                          nused - 1) * BM,
                         side="right"),
        0, E - 1).astype(jnp.int32)
    nused_arr = nused[None]

    gather_x, gather_y = _get_gathers()
    x_sorted = gather_x(x, tok_src)
    contrib = _grouped_ffn(block_expert, nused_arr, x_sorted, w1, w3, w2)
    y = gather_y(contrib, inv_pos).reshape(T, K * D)
    out = _combine(y, wts)
    return out.reshape(orig_shape)
